# baseline (speedup 1.0000x reference)
import jax
import jax.numpy as jnp
from jax import lax
from jax.experimental import pallas as pl
from jax.experimental.pallas import tpu as pltpu

NB = 512


def _chamfer_kernel(x_ref, y_ref, d1_ref, i1_ref, d2_ref, i2_ref):
    ni = pl.program_id(1)
    x = x_ref[0]
    y = y_ref[0]
    m = y.shape[1]

    x2 = jnp.sum(x * x, axis=1, keepdims=True)
    y2 = jnp.sum(y * y, axis=0, keepdims=True)
    c2 = jax.lax.dot_general(
        x + x, y, (((1,), (0,)), ((), ())),
        preferred_element_type=jnp.float32,
        precision=lax.Precision.HIGHEST)
    d = (x2 - c2) + y2

    m1 = jnp.min(d, axis=1, keepdims=True)
    jcol = lax.broadcasted_iota(jnp.int32, d.shape, 1)
    i1 = jnp.min(jnp.where(d == m1, jcol, m), axis=1, keepdims=True)
    d1_ref[0] = m1
    i1_ref[0] = i1

    m2 = jnp.min(d, axis=0, keepdims=True)
    riota = lax.broadcasted_iota(jnp.int32, d.shape, 0) + ni * NB
    i2 = jnp.min(jnp.where(d == m2, riota, jnp.int32(2**30)), axis=0,
                 keepdims=True)

    @pl.when(ni == 0)
    def _init():
        d2_ref[0] = m2
        i2_ref[0] = i2

    @pl.when(ni != 0)
    def _acc():
        prev_d = d2_ref[0]
        prev_i = i2_ref[0]
        upd = m2 < prev_d
        d2_ref[0] = jnp.where(upd, m2, prev_d)
        i2_ref[0] = jnp.where(upd, i2, prev_i)


def kernel(input1, input2):
    b, n, _ = input1.shape
    m = input2.shape[1]
    nblk = n // NB
    y_t = input2.transpose(0, 2, 1)

    d1, i1, d2, i2 = pl.pallas_call(
        _chamfer_kernel,
        grid=(b, nblk),
        in_specs=[
            pl.BlockSpec((1, NB, 3), lambda bi, ni: (bi, ni, 0)),
            pl.BlockSpec((1, 3, m), lambda bi, ni: (bi, 0, 0)),
        ],
        out_specs=[
            pl.BlockSpec((1, NB, 1), lambda bi, ni: (bi * nblk + ni, 0, 0)),
            pl.BlockSpec((1, NB, 1), lambda bi, ni: (bi * nblk + ni, 0, 0)),
            pl.BlockSpec((1, 1, m), lambda bi, ni: (bi, 0, 0)),
            pl.BlockSpec((1, 1, m), lambda bi, ni: (bi, 0, 0)),
        ],
        out_shape=[
            jax.ShapeDtypeStruct((b * nblk, NB, 1), jnp.float32),
            jax.ShapeDtypeStruct((b * nblk, NB, 1), jnp.int32),
            jax.ShapeDtypeStruct((b, 1, m), jnp.float32),
            jax.ShapeDtypeStruct((b, 1, m), jnp.int32),
        ],
        compiler_params=pltpu.CompilerParams(
            dimension_semantics=("parallel", "arbitrary")),
    )(input1, y_t)

    dist1 = d1.reshape(b, n)
    idx1 = i1.reshape(b, n)
    dist2 = d2.reshape(b, m)
    idx2 = i2.reshape(b, m)
    return (dist1, dist2, idx1, idx2)


# speedup vs baseline: 1.5556x; 1.5556x over previous
import jax
import jax.numpy as jnp
from jax import lax
from jax.experimental import pallas as pl
from jax.experimental.pallas import tpu as pltpu

NB = 512


def _chamfer_kernel(x_ref, y_ref, d1_ref, i1_ref, d2_ref, i2_ref):
    ni = pl.program_id(1)
    x = x_ref[0]
    y = y_ref[0]
    m = y.shape[1]

    d = (x[:, 0:1] - y[0:1, :]) ** 2
    d = d + (x[:, 1:2] - y[1:2, :]) ** 2
    d = d + (x[:, 2:3] - y[2:3, :]) ** 2

    big = jnp.float32(2**24)

    m1 = jnp.min(d, axis=1, keepdims=True)
    jcol = lax.broadcasted_iota(jnp.int32, (1, m), 1).astype(jnp.float32)
    i1f = jnp.min(jnp.where(d == m1, jcol, big), axis=1, keepdims=True)
    d1_ref[0] = m1
    i1_ref[0] = i1f.astype(jnp.int32)

    m2 = jnp.min(d, axis=0, keepdims=True)
    riota = (lax.broadcasted_iota(jnp.int32, (x.shape[0], 1), 0)
             + ni * NB).astype(jnp.float32)
    i2f = jnp.min(jnp.where(d == m2, riota, big), axis=0, keepdims=True)
    i2 = i2f.astype(jnp.int32)

    @pl.when(ni == 0)
    def _init():
        d2_ref[0] = m2
        i2_ref[0] = i2

    @pl.when(ni != 0)
    def _acc():
        prev_d = d2_ref[0]
        prev_i = i2_ref[0]
        upd = m2 < prev_d
        d2_ref[0] = jnp.where(upd, m2, prev_d)
        i2_ref[0] = jnp.where(upd, i2, prev_i)


def kernel(input1, input2):
    b, n, _ = input1.shape
    m = input2.shape[1]
    nblk = n // NB
    y_t = input2.transpose(0, 2, 1)

    d1, i1, d2, i2 = pl.pallas_call(
        _chamfer_kernel,
        grid=(b, nblk),
        in_specs=[
            pl.BlockSpec((1, NB, 3), lambda bi, ni: (bi, ni, 0)),
            pl.BlockSpec((1, 3, m), lambda bi, ni: (bi, 0, 0)),
        ],
        out_specs=[
            pl.BlockSpec((1, NB, 1), lambda bi, ni: (bi * nblk + ni, 0, 0)),
            pl.BlockSpec((1, NB, 1), lambda bi, ni: (bi * nblk + ni, 0, 0)),
            pl.BlockSpec((1, 1, m), lambda bi, ni: (bi, 0, 0)),
            pl.BlockSpec((1, 1, m), lambda bi, ni: (bi, 0, 0)),
        ],
        out_shape=[
            jax.ShapeDtypeStruct((b * nblk, NB, 1), jnp.float32),
            jax.ShapeDtypeStruct((b * nblk, NB, 1), jnp.int32),
            jax.ShapeDtypeStruct((b, 1, m), jnp.float32),
            jax.ShapeDtypeStruct((b, 1, m), jnp.int32),
        ],
        compiler_params=pltpu.CompilerParams(
            dimension_semantics=("parallel", "arbitrary")),
    )(input1, y_t)

    dist1 = d1.reshape(b, n)
    idx1 = i1.reshape(b, n)
    dist2 = d2.reshape(b, m)
    idx2 = i2.reshape(b, m)
    return (dist1, dist2, idx1, idx2)
